# baseline (speedup 1.0000x reference)
import jax
import jax.numpy as jnp
from jax.experimental import pallas as pl
from jax.experimental.pallas import tpu as pltpu

_ROWS = 128
_LATENT = 100000
_BLK = 8
_N = _ROWS // _BLK


def _gs_kernel(x_hbm, n_hbm, ret_hbm, lg_hbm,
               xb, nb, rb, lb, xs, ns, rs, ls):
    i = pl.program_id(0)
    slot = jax.lax.rem(i, 2)
    nxt = jax.lax.rem(i + 1, 2)

    def fetch(blk, p):
        r0 = blk * _BLK
        pltpu.make_async_copy(x_hbm.at[pl.ds(r0, _BLK), :],
                              xb.at[p], xs.at[p]).start()
        pltpu.make_async_copy(n_hbm.at[pl.ds(r0, _BLK), :],
                              nb.at[p], ns.at[p]).start()

    @pl.when(i == 0)
    def _():
        fetch(0, 0)

    @pl.when(i < _N - 1)
    def _():
        fetch(i + 1, nxt)

    r0 = i * _BLK
    pltpu.make_async_copy(x_hbm.at[pl.ds(r0, _BLK), :],
                          xb.at[slot], xs.at[slot]).wait()
    pltpu.make_async_copy(n_hbm.at[pl.ds(r0, _BLK), :],
                          nb.at[slot], ns.at[slot]).wait()

    @pl.when(i >= 2)
    def _():
        p0 = (i - 2) * _BLK
        pltpu.make_async_copy(rb.at[slot], ret_hbm.at[pl.ds(p0, _BLK), :],
                              rs.at[slot]).wait()
        pltpu.make_async_copy(lb.at[slot], lg_hbm.at[pl.ds(p0, _BLK), :],
                              ls.at[slot]).wait()

    x = xb[slot]
    m = jnp.max(x, axis=1, keepdims=True)
    s = jnp.sum(jnp.exp(x - m), axis=1, keepdims=True)
    lse = m + jnp.log(s)
    lg = x - lse
    g = lg + nb[slot]
    gmax = jnp.max(g, axis=1, keepdims=True)
    rb[slot] = (g == gmax).astype(x.dtype)
    lb[slot] = lg

    pltpu.make_async_copy(rb.at[slot], ret_hbm.at[pl.ds(r0, _BLK), :],
                          rs.at[slot]).start()
    pltpu.make_async_copy(lb.at[slot], lg_hbm.at[pl.ds(r0, _BLK), :],
                          ls.at[slot]).start()

    @pl.when(i == _N - 1)
    def _():
        p0 = (i - 1) * _BLK
        pltpu.make_async_copy(rb.at[nxt], ret_hbm.at[pl.ds(p0, _BLK), :],
                              rs.at[nxt]).wait()
        pltpu.make_async_copy(lb.at[nxt], lg_hbm.at[pl.ds(p0, _BLK), :],
                              ls.at[nxt]).wait()
        pltpu.make_async_copy(rb.at[slot], ret_hbm.at[pl.ds(r0, _BLK), :],
                              rs.at[slot]).wait()
        pltpu.make_async_copy(lb.at[slot], lg_hbm.at[pl.ds(r0, _BLK), :],
                              ls.at[slot]).wait()


def kernel(logits):
    noise = jax.random.gumbel(
        jax.random.key(42), (_ROWS, _LATENT), dtype=jnp.float32)
    ret, lg = pl.pallas_call(
        _gs_kernel,
        grid=(_N,),
        in_specs=[pl.BlockSpec(memory_space=pl.ANY)] * 2,
        out_specs=[pl.BlockSpec(memory_space=pl.ANY)] * 2,
        out_shape=[jax.ShapeDtypeStruct((_ROWS, _LATENT), jnp.float32)] * 2,
        scratch_shapes=[
            pltpu.VMEM((2, _BLK, _LATENT), jnp.float32),
            pltpu.VMEM((2, _BLK, _LATENT), jnp.float32),
            pltpu.VMEM((2, _BLK, _LATENT), jnp.float32),
            pltpu.VMEM((2, _BLK, _LATENT), jnp.float32),
            pltpu.SemaphoreType.DMA((2,)),
            pltpu.SemaphoreType.DMA((2,)),
            pltpu.SemaphoreType.DMA((2,)),
            pltpu.SemaphoreType.DMA((2,)),
        ],
    )(logits, noise)
    return ret, lg
